# baseline (speedup 1.0000x reference)
import functools

import jax
import jax.numpy as jnp
from jax import lax
from jax.experimental import pallas as pl
from jax.experimental.pallas import tpu as pltpu
from jax.experimental.pallas import tpu_sc as plsc

E = 8
K = 2
D = 1024
F = 2048
T = 2048
N = T * K

TM_R = 256
TM_C = 256
NT_C = N // TM_C
W_MAX = NT_C + E - 1
TM_D = 256

NW = 32
TPW = T // NW
P_C = 32

@functools.lru_cache(maxsize=None)
def _sc_mesh():
    return plsc.VectorSubcoreMesh(core_axis_name="c", subcore_axis_name="s")


def _router_body(x_ref, wr_ref, ws1_ref, ws3_ref, ws2_ref,
                 logits_ref, idx0_ref, idx1_ref, w0_ref, sel_ref, ysh_ref,
                 xbf_ref):
    x = x_ref[...]
    wr = wr_ref[...]
    xb = x.astype(jnp.bfloat16)
    xbf_ref[...] = xb
    h1 = lax.dot_general(xb, ws1_ref[...].astype(jnp.bfloat16),
                         (((1,), (1,)), ((), ())),
                         preferred_element_type=jnp.float32)
    h3 = lax.dot_general(xb, ws3_ref[...].astype(jnp.bfloat16),
                         (((1,), (1,)), ((), ())),
                         preferred_element_type=jnp.float32)
    h = (h1 * jax.nn.sigmoid(h1) * h3).astype(jnp.bfloat16)
    ysh_ref[...] = lax.dot_general(h, ws2_ref[...].astype(jnp.bfloat16),
                                   (((1,), (1,)), ((), ())),
                                   preferred_element_type=jnp.float32)
    logits = lax.dot_general(x, wr, (((1,), (1,)), ((), ())),
                             preferred_element_type=jnp.float32)
    logits_ref[...] = logits
    lane = lax.broadcasted_iota(jnp.int32, (TM_R, E), 1)
    m1 = jnp.max(logits, axis=1, keepdims=True)
    i1 = jnp.min(jnp.where(logits == m1, lane, E), axis=1, keepdims=True)
    masked = jnp.where(lane == i1, jnp.float32(-1e30), logits)
    m2 = jnp.max(masked, axis=1, keepdims=True)
    i2 = jnp.min(jnp.where(masked == m2, lane, E), axis=1, keepdims=True)
    w0_ref[...] = jnp.broadcast_to(jax.nn.sigmoid(m1 - m2), (TM_R, 16))
    idx0_ref[...] = i1
    idx1_ref[...] = i2
    sel_ref[...] = ((lane == i1) | (lane == i2)).astype(jnp.float32)


def _router(x, wr, ws1, ws3, ws2):
    return pl.pallas_call(
        _router_body,
        grid=(T // TM_R,),
        in_specs=[
            pl.BlockSpec((TM_R, D), lambda i: (i, 0)),
            pl.BlockSpec((E, D), lambda i: (0, 0)),
            pl.BlockSpec((F, D), lambda i: (0, 0)),
            pl.BlockSpec((F, D), lambda i: (0, 0)),
            pl.BlockSpec((D, F), lambda i: (0, 0)),
        ],
        out_specs=[
            pl.BlockSpec((TM_R, E), lambda i: (i, 0)),
            pl.BlockSpec((TM_R, 1), lambda i: (i, 0)),
            pl.BlockSpec((TM_R, 1), lambda i: (i, 0)),
            pl.BlockSpec((TM_R, 16), lambda i: (i, 0)),
            pl.BlockSpec((TM_R, E), lambda i: (i, 0)),
            pl.BlockSpec((TM_R, D), lambda i: (i, 0)),
            pl.BlockSpec((TM_R, D), lambda i: (i, 0)),
        ],
        out_shape=[
            jax.ShapeDtypeStruct((T, E), jnp.float32),
            jax.ShapeDtypeStruct((T, 1), jnp.int32),
            jax.ShapeDtypeStruct((T, 1), jnp.int32),
            jax.ShapeDtypeStruct((T, 16), jnp.float32),
            jax.ShapeDtypeStruct((T, E), jnp.float32),
            jax.ShapeDtypeStruct((T, D), jnp.float32),
            jax.ShapeDtypeStruct((T, D), jnp.bfloat16),
        ],
    )(x, wr, ws1, ws3, ws2)


def _rank_body(sel_ref, idx0_ref, idx1_ref, c_ref, cg0_ref, cg1_ref):
    m = pl.program_id(0)
    sel = sel_ref[...]
    row = lax.broadcasted_iota(jnp.int32, (TM_R, T), 0) + m * TM_R
    col = lax.broadcasted_iota(jnp.int32, (TM_R, T), 1)
    ltri = (row > col).astype(jnp.float32)
    c = lax.dot_general(ltri, sel, (((1,), (0,)), ((), ())),
                        preferred_element_type=jnp.float32)
    c_ref[...] = c
    lane = lax.broadcasted_iota(jnp.int32, (TM_R, E), 1)
    i0 = idx0_ref[...]
    i1 = idx1_ref[...]
    cg0_ref[...] = jnp.sum(jnp.where(lane == i0, c, 0.0), axis=1,
                           keepdims=True).astype(jnp.int32)
    cg1_ref[...] = jnp.sum(jnp.where(lane == i1, c, 0.0), axis=1,
                           keepdims=True).astype(jnp.int32)


def _ranks(sel, idx0, idx1):
    return pl.pallas_call(
        _rank_body,
        grid=(T // TM_R,),
        in_specs=[
            pl.BlockSpec((T, E), lambda i: (0, 0)),
            pl.BlockSpec((TM_R, 1), lambda i: (i, 0)),
            pl.BlockSpec((TM_R, 1), lambda i: (i, 0)),
        ],
        out_specs=[
            pl.BlockSpec((TM_R, E), lambda i: (i, 0)),
            pl.BlockSpec((TM_R, 1), lambda i: (i, 0)),
            pl.BlockSpec((TM_R, 1), lambda i: (i, 0)),
        ],
        out_shape=[
            jax.ShapeDtypeStruct((T, E), jnp.float32),
            jax.ShapeDtypeStruct((T, 1), jnp.int32),
            jax.ShapeDtypeStruct((T, 1), jnp.int32),
        ],
    )(sel, idx0, idx1)


def _dispatch_body(x_hbm, pos_hbm, xs_hbm, rows_v, idx0_v, idx1_v, sem):
    wid = lax.axis_index("s") * 2 + lax.axis_index("c")
    base = wid * TPW
    pltpu.sync_copy(x_hbm.at[pl.ds(base, TPW)], rows_v)
    pltpu.sync_copy(pos_hbm.at[wid, 0], idx0_v)
    pltpu.sync_copy(pos_hbm.at[wid, 1], idx1_v)
    cp0 = pltpu.async_copy(rows_v, xs_hbm.at[idx0_v], sem)
    cp1 = pltpu.async_copy(rows_v, xs_hbm.at[idx1_v], sem)
    cp0.wait()
    cp1.wait()


@functools.lru_cache(maxsize=None)
def _make_dispatch_sc():
    return pl.kernel(
        _dispatch_body,
        out_type=jax.ShapeDtypeStruct((N, D // 2), jnp.int32),
        mesh=_sc_mesh(),
        scratch_types=[
            pltpu.VMEM((TPW, D // 2), jnp.int32),
            pltpu.VMEM((TPW,), jnp.int32),
            pltpu.VMEM((TPW,), jnp.int32),
            pltpu.SemaphoreType.DMA,
        ],
    )


def _dispatch_sc(x, pos):
    return _make_dispatch_sc()(x, pos)


def _gmm_body(meta_ref, xs_ref, w1_ref, w3_ref, w2_ref, out_ref):
    i = pl.program_id(0)
    m = meta_ref[0, i]
    lo = meta_ref[2, i]
    hi = meta_ref[3, i]
    first = meta_ref[4, i]
    x = xs_ref[...]
    w1 = w1_ref[0].astype(jnp.bfloat16)
    w3 = w3_ref[0].astype(jnp.bfloat16)
    w2 = w2_ref[0]
    h1 = lax.dot_general(x, w1, (((1,), (1,)), ((), ())),
                         preferred_element_type=jnp.float32)
    h3 = lax.dot_general(x, w3, (((1,), (1,)), ((), ())),
                         preferred_element_type=jnp.float32)
    h = h1 * jax.nn.sigmoid(h1) * h3
    y = lax.dot_general(h, w2, (((1,), (1,)), ((), ())),
                        preferred_element_type=jnp.float32)
    rows = lax.broadcasted_iota(jnp.int32, (TM_C, 1), 0) + m * TM_C
    y = jnp.where((rows >= lo) & (rows < hi), y, 0.0)

    @pl.when(first == 1)
    def _():
        out_ref[...] = y

    @pl.when(first == 0)
    def _():
        out_ref[...] += y


def _gmm(meta, xs, w1, w3, w2):
    grid_spec = pltpu.PrefetchScalarGridSpec(
        num_scalar_prefetch=1,
        grid=(W_MAX,),
        in_specs=[
            pl.BlockSpec((TM_C, D), lambda i, m: (m[0, i], 0)),
            pl.BlockSpec((1, F, D), lambda i, m: (m[1, i], 0, 0)),
            pl.BlockSpec((1, F, D), lambda i, m: (m[1, i], 0, 0)),
            pl.BlockSpec((1, D, F), lambda i, m: (m[1, i], 0, 0)),
        ],
        out_specs=pl.BlockSpec((TM_C, D), lambda i, m: (m[0, i], 0)),
    )
    return pl.pallas_call(
        _gmm_body,
        grid_spec=grid_spec,
        out_shape=jax.ShapeDtypeStruct((N, D), jnp.float32),
    )(meta, xs, w1, w3, w2)


P_G = 32


def _combine_body(ys_hbm, ysh_hbm, w0_hbm, pos_hbm, out_hbm,
                  i_v, g0_v, g1_v, out_v, w_v, sem):
    wid = lax.axis_index("s") * 2 + lax.axis_index("c")
    for p in range(TPW // P_G):
        base = wid * TPW + p * P_G
        pltpu.sync_copy(pos_hbm.at[wid, 0, pl.ds(p * P_G, P_G)], i_v)
        pltpu.async_copy(ys_hbm.at[i_v], g0_v, sem).wait()
        pltpu.sync_copy(pos_hbm.at[wid, 1, pl.ds(p * P_G, P_G)], i_v)
        pltpu.async_copy(ys_hbm.at[i_v], g1_v, sem).wait()
        pltpu.sync_copy(ysh_hbm.at[pl.ds(base, P_G)], out_v)
        pltpu.sync_copy(w0_hbm.at[pl.ds(base, P_G)], w_v)

        def row_body(r, carry):
            s0 = w_v[r]
            s1 = 1.0 - s0
            for c in range(D // 16):
                sl = pl.ds(c * 16, 16)
                out_v[r, sl] = (out_v[r, sl] + g0_v[r, sl] * s0
                                + g1_v[r, sl] * s1)
            return carry

        lax.fori_loop(0, P_G, row_body, 0)
        pltpu.sync_copy(out_v, out_hbm.at[pl.ds(base, P_G)])


@functools.lru_cache(maxsize=None)
def _make_combine_sc():
    return pl.kernel(
        _combine_body,
        out_type=jax.ShapeDtypeStruct((T, D), jnp.float32),
        mesh=_sc_mesh(),
        scratch_types=[
            pltpu.VMEM((P_G,), jnp.int32),
            pltpu.VMEM((P_G, D), jnp.float32),
            pltpu.VMEM((P_G, D), jnp.float32),
            pltpu.VMEM((P_G, D), jnp.float32),
            pltpu.VMEM((P_G, 16), jnp.float32),
            pltpu.SemaphoreType.DMA,
        ],
    )


def _combine_sc(ys, ysh, w0b, pos):
    return _make_combine_sc()(ys, ysh, w0b, pos)


def _build_meta(counts, offs):
    m_grid = jnp.arange(NT_C, dtype=jnp.int32)[:, None]
    e_grid = jnp.arange(E, dtype=jnp.int32)[None, :]
    start = offs[None, :]
    end = (offs + counts)[None, :]
    lo = jnp.maximum(start, m_grid * TM_C)
    hi = jnp.minimum(end, (m_grid + 1) * TM_C)
    hit = (lo < hi).ravel()
    ar = jnp.arange(NT_C * E, dtype=jnp.int32)
    order = jnp.argsort(jnp.where(hit, ar, NT_C * E + ar))
    nh = jnp.sum(hit.astype(jnp.int32))
    src = order[jnp.minimum(jnp.arange(W_MAX, dtype=jnp.int32), nh - 1)]
    valid = jnp.arange(W_MAX, dtype=jnp.int32) < nh
    m_of = jnp.broadcast_to(m_grid, (NT_C, E)).ravel()[src]
    e_of = jnp.broadcast_to(e_grid, (NT_C, E)).ravel()[src]
    lo_j = jnp.where(valid, lo.ravel()[src], 0)
    hi_j = jnp.where(valid, hi.ravel()[src], 0)
    first = jnp.concatenate(
        [jnp.ones((1,), jnp.int32),
         (m_of[1:] != m_of[:-1]).astype(jnp.int32)])
    return jnp.stack([m_of, e_of, lo_j, hi_j, first]).astype(jnp.int32)


def kernel(hidden_states, Wr, W1, W3, W2, Ws1, Ws3, Ws2):
    b, s, d = hidden_states.shape
    x = hidden_states.reshape(T, D)
    logits, idx0c, idx1c, w0b, sel, ysh, xbf = _router(x, Wr, Ws1, Ws3, Ws2)
    cfull, cg0, cg1 = _ranks(sel, idx0c, idx1c)
    counts = (cfull[-1] + sel[-1]).astype(jnp.int32)
    offs = jnp.concatenate(
        [jnp.zeros((1,), jnp.int32), jnp.cumsum(counts)[:-1]])
    idx0 = idx0c[:, 0]
    idx1 = idx1c[:, 0]
    pos0 = cg0[:, 0] + jnp.take(offs, idx0)
    pos1 = cg1[:, 0] + jnp.take(offs, idx1)
    pos = jnp.stack(
        [pos0.reshape(NW, TPW), pos1.reshape(NW, TPW)], axis=1)
    x32 = jax.lax.bitcast_convert_type(
        xbf.reshape(T, D // 2, 2), jnp.int32)
    xs32 = _dispatch_sc(x32, pos)
    xs = jax.lax.bitcast_convert_type(xs32, jnp.bfloat16).reshape(N, D)
    meta = _build_meta(counts, offs)
    ys = _gmm(meta, xs, W1, W3, W2)
    out = _combine_sc(ys, ysh, w0b, pos)
    return out.reshape(b, s, d), logits


# speedup vs baseline: 1.4259x; 1.4259x over previous
import functools

import jax
import jax.numpy as jnp
from jax import lax
from jax.experimental import pallas as pl
from jax.experimental.pallas import tpu as pltpu
from jax.experimental.pallas import tpu_sc as plsc

E = 8
K = 2
D = 1024
F = 2048
T = 2048
N = T * K

TM_R = 256
TM_C = 256
NT_C = N // TM_C
W_MAX = NT_C + E - 1
TM_D = 256

NW = 32
TPW = T // NW
P_C = 32

@functools.lru_cache(maxsize=None)
def _sc_mesh():
    return plsc.VectorSubcoreMesh(core_axis_name="c", subcore_axis_name="s")


def _router_body(x_ref, wr_ref, ws1_ref, ws3_ref, ws2_ref,
                 logits_ref, idx0_ref, idx1_ref, w0_ref, sel_ref, ysh_ref):
    x = x_ref[...]
    wr = wr_ref[...]
    h1 = lax.dot_general(x, ws1_ref[...], (((1,), (1,)), ((), ())),
                         preferred_element_type=jnp.float32)
    h3 = lax.dot_general(x, ws3_ref[...], (((1,), (1,)), ((), ())),
                         preferred_element_type=jnp.float32)
    h = h1 * jax.nn.sigmoid(h1) * h3
    ysh_ref[...] = lax.dot_general(h, ws2_ref[...], (((1,), (1,)), ((), ())),
                                   preferred_element_type=jnp.float32)
    logits = lax.dot_general(x, wr, (((1,), (1,)), ((), ())),
                             preferred_element_type=jnp.float32)
    logits_ref[...] = logits
    lane = lax.broadcasted_iota(jnp.int32, (TM_R, E), 1)
    m1 = jnp.max(logits, axis=1, keepdims=True)
    i1 = jnp.min(jnp.where(logits == m1, lane, E), axis=1, keepdims=True)
    masked = jnp.where(lane == i1, jnp.float32(-1e30), logits)
    m2 = jnp.max(masked, axis=1, keepdims=True)
    i2 = jnp.min(jnp.where(masked == m2, lane, E), axis=1, keepdims=True)
    w0_ref[...] = jnp.broadcast_to(jax.nn.sigmoid(m1 - m2), (TM_R, 16))
    idx0_ref[...] = i1
    idx1_ref[...] = i2
    sel_ref[...] = ((lane == i1) | (lane == i2)).astype(jnp.float32)


def _router(x, wr, ws1, ws3, ws2):
    return pl.pallas_call(
        _router_body,
        grid=(T // TM_R,),
        in_specs=[
            pl.BlockSpec((TM_R, D), lambda i: (i, 0)),
            pl.BlockSpec((E, D), lambda i: (0, 0)),
            pl.BlockSpec((F, D), lambda i: (0, 0)),
            pl.BlockSpec((F, D), lambda i: (0, 0)),
            pl.BlockSpec((D, F), lambda i: (0, 0)),
        ],
        out_specs=[
            pl.BlockSpec((TM_R, E), lambda i: (i, 0)),
            pl.BlockSpec((TM_R, 1), lambda i: (i, 0)),
            pl.BlockSpec((TM_R, 1), lambda i: (i, 0)),
            pl.BlockSpec((TM_R, 16), lambda i: (i, 0)),
            pl.BlockSpec((TM_R, E), lambda i: (i, 0)),
            pl.BlockSpec((TM_R, D), lambda i: (i, 0)),
        ],
        out_shape=[
            jax.ShapeDtypeStruct((T, E), jnp.float32),
            jax.ShapeDtypeStruct((T, 1), jnp.int32),
            jax.ShapeDtypeStruct((T, 1), jnp.int32),
            jax.ShapeDtypeStruct((T, 16), jnp.float32),
            jax.ShapeDtypeStruct((T, E), jnp.float32),
            jax.ShapeDtypeStruct((T, D), jnp.float32),
        ],
    )(x, wr, ws1, ws3, ws2)


def _rank_body(sel_ref, idx0_ref, idx1_ref, c_ref, cg0_ref, cg1_ref):
    m = pl.program_id(0)
    sel = sel_ref[...]
    row = lax.broadcasted_iota(jnp.int32, (TM_R, T), 0) + m * TM_R
    col = lax.broadcasted_iota(jnp.int32, (TM_R, T), 1)
    ltri = (row > col).astype(jnp.float32)
    c = lax.dot_general(ltri, sel, (((1,), (0,)), ((), ())),
                        preferred_element_type=jnp.float32)
    c_ref[...] = c
    lane = lax.broadcasted_iota(jnp.int32, (TM_R, E), 1)
    i0 = idx0_ref[...]
    i1 = idx1_ref[...]
    cg0_ref[...] = jnp.sum(jnp.where(lane == i0, c, 0.0), axis=1,
                           keepdims=True).astype(jnp.int32)
    cg1_ref[...] = jnp.sum(jnp.where(lane == i1, c, 0.0), axis=1,
                           keepdims=True).astype(jnp.int32)


def _ranks(sel, idx0, idx1):
    return pl.pallas_call(
        _rank_body,
        grid=(T // TM_R,),
        in_specs=[
            pl.BlockSpec((T, E), lambda i: (0, 0)),
            pl.BlockSpec((TM_R, 1), lambda i: (i, 0)),
            pl.BlockSpec((TM_R, 1), lambda i: (i, 0)),
        ],
        out_specs=[
            pl.BlockSpec((TM_R, E), lambda i: (i, 0)),
            pl.BlockSpec((TM_R, 1), lambda i: (i, 0)),
            pl.BlockSpec((TM_R, 1), lambda i: (i, 0)),
        ],
        out_shape=[
            jax.ShapeDtypeStruct((T, E), jnp.float32),
            jax.ShapeDtypeStruct((T, 1), jnp.int32),
            jax.ShapeDtypeStruct((T, 1), jnp.int32),
        ],
    )(sel, idx0, idx1)


def _dispatch_body(x_hbm, pos_hbm, xs_hbm, rows_v, idx0_v, idx1_v, sem):
    wid = lax.axis_index("s") * 2 + lax.axis_index("c")
    base = wid * TPW
    pltpu.sync_copy(x_hbm.at[pl.ds(base, TPW)], rows_v)
    pltpu.sync_copy(pos_hbm.at[wid, 0], idx0_v)
    pltpu.sync_copy(pos_hbm.at[wid, 1], idx1_v)
    cp0 = pltpu.async_copy(rows_v, xs_hbm.at[idx0_v], sem)
    cp1 = pltpu.async_copy(rows_v, xs_hbm.at[idx1_v], sem)
    cp0.wait()
    cp1.wait()


@functools.lru_cache(maxsize=None)
def _make_dispatch_sc():
    return pl.kernel(
        _dispatch_body,
        out_type=jax.ShapeDtypeStruct((N, D), jnp.float32),
        mesh=_sc_mesh(),
        scratch_types=[
            pltpu.VMEM((TPW, D), jnp.float32),
            pltpu.VMEM((TPW,), jnp.int32),
            pltpu.VMEM((TPW,), jnp.int32),
            pltpu.SemaphoreType.DMA,
        ],
    )


def _dispatch_sc(x, pos):
    return _make_dispatch_sc()(x, pos)


def _gmm_body(meta_ref, xs_ref, w1_ref, w3_ref, w2_ref, out_ref):
    i = pl.program_id(0)
    m = meta_ref[0, i]
    lo = meta_ref[2, i]
    hi = meta_ref[3, i]
    first = meta_ref[4, i]
    x = xs_ref[...]
    w1 = w1_ref[0]
    w3 = w3_ref[0]
    w2 = w2_ref[0]
    h1 = lax.dot_general(x, w1, (((1,), (1,)), ((), ())),
                         preferred_element_type=jnp.float32)
    h3 = lax.dot_general(x, w3, (((1,), (1,)), ((), ())),
                         preferred_element_type=jnp.float32)
    h = h1 * jax.nn.sigmoid(h1) * h3
    y = lax.dot_general(h, w2, (((1,), (1,)), ((), ())),
                        preferred_element_type=jnp.float32)
    rows = lax.broadcasted_iota(jnp.int32, (TM_C, 1), 0) + m * TM_C
    y = jnp.where((rows >= lo) & (rows < hi), y, 0.0)

    @pl.when(first == 1)
    def _():
        out_ref[...] = y

    @pl.when(first == 0)
    def _():
        out_ref[...] += y


def _gmm(meta, xs, w1, w3, w2):
    grid_spec = pltpu.PrefetchScalarGridSpec(
        num_scalar_prefetch=1,
        grid=(W_MAX,),
        in_specs=[
            pl.BlockSpec((TM_C, D), lambda i, m: (m[0, i], 0)),
            pl.BlockSpec((1, F, D), lambda i, m: (m[1, i], 0, 0)),
            pl.BlockSpec((1, F, D), lambda i, m: (m[1, i], 0, 0)),
            pl.BlockSpec((1, D, F), lambda i, m: (m[1, i], 0, 0)),
        ],
        out_specs=pl.BlockSpec((TM_C, D), lambda i, m: (m[0, i], 0)),
    )
    return pl.pallas_call(
        _gmm_body,
        grid_spec=grid_spec,
        out_shape=jax.ShapeDtypeStruct((N, D), jnp.float32),
    )(meta, xs, w1, w3, w2)


P_G = 16


def _combine_body(ys_hbm, ysh_hbm, w0_hbm, pos_hbm, out_hbm,
                  i0_v, i1_v, g0_v, g1_v, out_v, w_v,
                  sg0a, sg0b, sg1a, sg1b, soa, sob, swa, swb):
    wid = lax.axis_index("s") * 2 + lax.axis_index("c")
    np_ = TPW // P_G
    sg0 = (sg0a, sg0b)
    sg1 = (sg1a, sg1b)
    so = (soa, sob)
    sw = (swa, swb)
    wrcp = [None, None]

    def issue(p):
        b = p % 2
        base = wid * TPW + p * P_G
        pltpu.sync_copy(pos_hbm.at[wid, 0, pl.ds(p * P_G, P_G)], i0_v.at[b])
        pltpu.sync_copy(pos_hbm.at[wid, 1, pl.ds(p * P_G, P_G)], i1_v.at[b])
        c0 = pltpu.async_copy(ys_hbm.at[i0_v.at[b]], g0_v.at[b], sg0[b])
        c1 = pltpu.async_copy(ys_hbm.at[i1_v.at[b]], g1_v.at[b], sg1[b])
        co = pltpu.async_copy(ysh_hbm.at[pl.ds(base, P_G)], out_v.at[b],
                              so[b])
        pltpu.sync_copy(w0_hbm.at[pl.ds(base, P_G)], w_v.at[b])
        return (c0, c1, co)

    pend = issue(0)
    for p in range(np_):
        b = p % 2
        nxt = None
        if p + 1 < np_:
            if wrcp[(p + 1) % 2] is not None:
                wrcp[(p + 1) % 2].wait()
            nxt = issue(p + 1)
        c0, c1, co = pend
        c0.wait()
        c1.wait()
        co.wait()

        def row_body(r, carry):
            s0 = w_v[b, r]
            s1 = 1.0 - s0
            for c in range(D // 16):
                sl = pl.ds(c * 16, 16)
                out_v[b, r, sl] = (out_v[b, r, sl] + g0_v[b, r, sl] * s0
                                   + g1_v[b, r, sl] * s1)
            return carry

        lax.fori_loop(0, P_G, row_body, 0)
        wrcp[b] = pltpu.async_copy(
            out_v.at[b], out_hbm.at[pl.ds(wid * TPW + p * P_G, P_G)], sw[b])
        pend = nxt
    wrcp[0].wait()
    wrcp[1].wait()


@functools.lru_cache(maxsize=None)
def _make_combine_sc():
    return pl.kernel(
        _combine_body,
        out_type=jax.ShapeDtypeStruct((T, D), jnp.float32),
        mesh=_sc_mesh(),
        scratch_types=[
            pltpu.VMEM((2, P_G), jnp.int32),
            pltpu.VMEM((2, P_G), jnp.int32),
            pltpu.VMEM((2, P_G, D), jnp.float32),
            pltpu.VMEM((2, P_G, D), jnp.float32),
            pltpu.VMEM((2, P_G, D), jnp.float32),
            pltpu.VMEM((2, P_G, 16), jnp.float32),
            pltpu.SemaphoreType.DMA,
            pltpu.SemaphoreType.DMA,
            pltpu.SemaphoreType.DMA,
            pltpu.SemaphoreType.DMA,
            pltpu.SemaphoreType.DMA,
            pltpu.SemaphoreType.DMA,
            pltpu.SemaphoreType.DMA,
            pltpu.SemaphoreType.DMA,
        ],
    )


def _combine_sc(ys, ysh, w0b, pos):
    return _make_combine_sc()(ys, ysh, w0b, pos)


def _build_meta(counts, offs):
    m_grid = jnp.arange(NT_C, dtype=jnp.int32)[:, None]
    e_grid = jnp.arange(E, dtype=jnp.int32)[None, :]
    start = offs[None, :]
    end = (offs + counts)[None, :]
    lo = jnp.maximum(start, m_grid * TM_C)
    hi = jnp.minimum(end, (m_grid + 1) * TM_C)
    hit = (lo < hi).ravel()
    ar = jnp.arange(NT_C * E, dtype=jnp.int32)
    order = jnp.argsort(jnp.where(hit, ar, NT_C * E + ar))
    nh = jnp.sum(hit.astype(jnp.int32))
    src = order[jnp.minimum(jnp.arange(W_MAX, dtype=jnp.int32), nh - 1)]
    valid = jnp.arange(W_MAX, dtype=jnp.int32) < nh
    m_of = jnp.broadcast_to(m_grid, (NT_C, E)).ravel()[src]
    e_of = jnp.broadcast_to(e_grid, (NT_C, E)).ravel()[src]
    lo_j = jnp.where(valid, lo.ravel()[src], 0)
    hi_j = jnp.where(valid, hi.ravel()[src], 0)
    first = jnp.concatenate(
        [jnp.ones((1,), jnp.int32),
         (m_of[1:] != m_of[:-1]).astype(jnp.int32)])
    return jnp.stack([m_of, e_of, lo_j, hi_j, first]).astype(jnp.int32)


def kernel(hidden_states, Wr, W1, W3, W2, Ws1, Ws3, Ws2):
    b, s, d = hidden_states.shape
    x = hidden_states.reshape(T, D)
    logits, idx0c, idx1c, w0b, sel, ysh = _router(x, Wr, Ws1, Ws3, Ws2)
    cfull, cg0, cg1 = _ranks(sel, idx0c, idx1c)
    counts = (cfull[-1] + sel[-1]).astype(jnp.int32)
    offs = jnp.concatenate(
        [jnp.zeros((1,), jnp.int32), jnp.cumsum(counts)[:-1]])
    idx0 = idx0c[:, 0]
    idx1 = idx1c[:, 0]
    pos0 = cg0[:, 0] + jnp.take(offs, idx0)
    pos1 = cg1[:, 0] + jnp.take(offs, idx1)
    pos = jnp.stack(
        [pos0.reshape(NW, TPW), pos1.reshape(NW, TPW)], axis=1)
    xs = _dispatch_sc(x, pos)
    meta = _build_meta(counts, offs)
    ys = _gmm(meta, xs, W1, W3, W2)
    out = _combine_sc(ys, ysh, w0b, pos)
    return out.reshape(b, s, d), logits


# speedup vs baseline: 1.4800x; 1.0379x over previous
import functools

import jax
import jax.numpy as jnp
from jax import lax
from jax.experimental import pallas as pl
from jax.experimental.pallas import tpu as pltpu
from jax.experimental.pallas import tpu_sc as plsc

E = 8
K = 2
D = 1024
F = 2048
T = 2048
N = T * K

TM_R = 256
TM_C = 256
NT_C = N // TM_C
W_MAX = NT_C + E - 1
TM_D = 256

NW = 32
TPW = T // NW
P_C = 32

@functools.lru_cache(maxsize=None)
def _sc_mesh():
    return plsc.VectorSubcoreMesh(core_axis_name="c", subcore_axis_name="s")


def _router_body(x_ref, wr_ref, logits_ref, idx0_ref, idx1_ref, w0_ref,
                 sel_ref):
    x = x_ref[...]
    wr = wr_ref[...]
    logits = lax.dot_general(x, wr, (((1,), (1,)), ((), ())),
                             preferred_element_type=jnp.float32)
    logits_ref[...] = logits
    lane = lax.broadcasted_iota(jnp.int32, (TM_R, E), 1)
    m1 = jnp.max(logits, axis=1, keepdims=True)
    i1 = jnp.min(jnp.where(logits == m1, lane, E), axis=1, keepdims=True)
    masked = jnp.where(lane == i1, jnp.float32(-1e30), logits)
    m2 = jnp.max(masked, axis=1, keepdims=True)
    i2 = jnp.min(jnp.where(masked == m2, lane, E), axis=1, keepdims=True)
    w0_ref[...] = jnp.broadcast_to(jax.nn.sigmoid(m1 - m2), (TM_R, 16))
    idx0_ref[...] = i1
    idx1_ref[...] = i2
    sel_ref[...] = ((lane == i1) | (lane == i2)).astype(jnp.float32)


def _router(x, wr):
    return pl.pallas_call(
        _router_body,
        grid=(T // TM_R,),
        in_specs=[
            pl.BlockSpec((TM_R, D), lambda i: (i, 0)),
            pl.BlockSpec((E, D), lambda i: (0, 0)),
        ],
        out_specs=[
            pl.BlockSpec((TM_R, E), lambda i: (i, 0)),
            pl.BlockSpec((TM_R, 1), lambda i: (i, 0)),
            pl.BlockSpec((TM_R, 1), lambda i: (i, 0)),
            pl.BlockSpec((TM_R, 16), lambda i: (i, 0)),
            pl.BlockSpec((TM_R, E), lambda i: (i, 0)),
        ],
        out_shape=[
            jax.ShapeDtypeStruct((T, E), jnp.float32),
            jax.ShapeDtypeStruct((T, 1), jnp.int32),
            jax.ShapeDtypeStruct((T, 1), jnp.int32),
            jax.ShapeDtypeStruct((T, 16), jnp.float32),
            jax.ShapeDtypeStruct((T, E), jnp.float32),
        ],
    )(x, wr)


def _shared_body(x_ref, w1_ref, w3_ref, w2_ref, out_ref):
    x = x_ref[...]
    h1 = lax.dot_general(x, w1_ref[...], (((1,), (1,)), ((), ())),
                         preferred_element_type=jnp.float32)
    h3 = lax.dot_general(x, w3_ref[...], (((1,), (1,)), ((), ())),
                         preferred_element_type=jnp.float32)
    h = h1 * jax.nn.sigmoid(h1) * h3
    out_ref[...] = lax.dot_general(h, w2_ref[...], (((1,), (1,)), ((), ())),
                                   preferred_element_type=jnp.float32)


def _shared(x, ws1, ws3, ws2):
    return pl.pallas_call(
        _shared_body,
        grid=(T // TM_D,),
        in_specs=[
            pl.BlockSpec((TM_D, D), lambda i: (i, 0)),
            pl.BlockSpec((F, D), lambda i: (0, 0)),
            pl.BlockSpec((F, D), lambda i: (0, 0)),
            pl.BlockSpec((D, F), lambda i: (0, 0)),
        ],
        out_specs=pl.BlockSpec((TM_D, D), lambda i: (i, 0)),
        out_shape=jax.ShapeDtypeStruct((T, D), jnp.float32),
    )(x, ws1, ws3, ws2)


def _mix_body(part_ref, ysh_ref, out_ref):
    out_ref[...] = part_ref[...] + ysh_ref[...]


def _mix(part, ysh):
    return pl.pallas_call(
        _mix_body,
        grid=(T // TM_D,),
        in_specs=[
            pl.BlockSpec((TM_D, D), lambda i: (i, 0)),
            pl.BlockSpec((TM_D, D), lambda i: (i, 0)),
        ],
        out_specs=pl.BlockSpec((TM_D, D), lambda i: (i, 0)),
        out_shape=jax.ShapeDtypeStruct((T, D), jnp.float32),
    )(part, ysh)


def _rank_body(sel_ref, idx0_ref, idx1_ref, c_ref, cg0_ref, cg1_ref):
    m = pl.program_id(0)
    sel = sel_ref[...]
    row = lax.broadcasted_iota(jnp.int32, (TM_R, T), 0) + m * TM_R
    col = lax.broadcasted_iota(jnp.int32, (TM_R, T), 1)
    ltri = (row > col).astype(jnp.float32)
    c = lax.dot_general(ltri, sel, (((1,), (0,)), ((), ())),
                        preferred_element_type=jnp.float32)
    c_ref[...] = c
    lane = lax.broadcasted_iota(jnp.int32, (TM_R, E), 1)
    i0 = idx0_ref[...]
    i1 = idx1_ref[...]
    cg0_ref[...] = jnp.sum(jnp.where(lane == i0, c, 0.0), axis=1,
                           keepdims=True).astype(jnp.int32)
    cg1_ref[...] = jnp.sum(jnp.where(lane == i1, c, 0.0), axis=1,
                           keepdims=True).astype(jnp.int32)


def _ranks(sel, idx0, idx1):
    return pl.pallas_call(
        _rank_body,
        grid=(T // TM_R,),
        in_specs=[
            pl.BlockSpec((T, E), lambda i: (0, 0)),
            pl.BlockSpec((TM_R, 1), lambda i: (i, 0)),
            pl.BlockSpec((TM_R, 1), lambda i: (i, 0)),
        ],
        out_specs=[
            pl.BlockSpec((TM_R, E), lambda i: (i, 0)),
            pl.BlockSpec((TM_R, 1), lambda i: (i, 0)),
            pl.BlockSpec((TM_R, 1), lambda i: (i, 0)),
        ],
        out_shape=[
            jax.ShapeDtypeStruct((T, E), jnp.float32),
            jax.ShapeDtypeStruct((T, 1), jnp.int32),
            jax.ShapeDtypeStruct((T, 1), jnp.int32),
        ],
    )(sel, idx0, idx1)


def _dispatch_body(x_hbm, pos_hbm, xs_hbm, rows_v, idx0_v, idx1_v, sem):
    wid = lax.axis_index("s") * 2 + lax.axis_index("c")
    base = wid * TPW
    pltpu.sync_copy(x_hbm.at[pl.ds(base, TPW)], rows_v)
    pltpu.sync_copy(pos_hbm.at[wid, 0], idx0_v)
    pltpu.sync_copy(pos_hbm.at[wid, 1], idx1_v)
    cp0 = pltpu.async_copy(rows_v, xs_hbm.at[idx0_v], sem)
    cp1 = pltpu.async_copy(rows_v, xs_hbm.at[idx1_v], sem)
    cp0.wait()
    cp1.wait()


@functools.lru_cache(maxsize=None)
def _make_dispatch_sc():
    return pl.kernel(
        _dispatch_body,
        out_type=jax.ShapeDtypeStruct((N, D), jnp.float32),
        mesh=_sc_mesh(),
        scratch_types=[
            pltpu.VMEM((TPW, D), jnp.float32),
            pltpu.VMEM((TPW,), jnp.int32),
            pltpu.VMEM((TPW,), jnp.int32),
            pltpu.SemaphoreType.DMA,
        ],
    )


def _dispatch_sc(x, pos):
    return _make_dispatch_sc()(x, pos)


def _gmm_body(meta_ref, xs_ref, w1_ref, w3_ref, w2_ref, out_ref):
    i = pl.program_id(0)
    m = meta_ref[0, i]
    lo = meta_ref[2, i]
    hi = meta_ref[3, i]
    first = meta_ref[4, i]
    x = xs_ref[...]
    w1 = w1_ref[0]
    w3 = w3_ref[0]
    w2 = w2_ref[0]
    h1 = lax.dot_general(x, w1, (((1,), (1,)), ((), ())),
                         preferred_element_type=jnp.float32)
    h3 = lax.dot_general(x, w3, (((1,), (1,)), ((), ())),
                         preferred_element_type=jnp.float32)
    h = h1 * jax.nn.sigmoid(h1) * h3
    y = lax.dot_general(h, w2, (((1,), (1,)), ((), ())),
                        preferred_element_type=jnp.float32)
    rows = lax.broadcasted_iota(jnp.int32, (TM_C, 1), 0) + m * TM_C
    y = jnp.where((rows >= lo) & (rows < hi), y, 0.0)

    @pl.when(first == 1)
    def _():
        out_ref[...] = y

    @pl.when(first == 0)
    def _():
        out_ref[...] += y


def _gmm(meta, xs, w1, w3, w2):
    grid_spec = pltpu.PrefetchScalarGridSpec(
        num_scalar_prefetch=1,
        grid=(W_MAX,),
        in_specs=[
            pl.BlockSpec((TM_C, D), lambda i, m: (m[0, i], 0)),
            pl.BlockSpec((1, F, D), lambda i, m: (m[1, i], 0, 0)),
            pl.BlockSpec((1, F, D), lambda i, m: (m[1, i], 0, 0)),
            pl.BlockSpec((1, D, F), lambda i, m: (m[1, i], 0, 0)),
        ],
        out_specs=pl.BlockSpec((TM_C, D), lambda i, m: (m[0, i], 0)),
    )
    return pl.pallas_call(
        _gmm_body,
        grid_spec=grid_spec,
        out_shape=jax.ShapeDtypeStruct((N, D), jnp.float32),
    )(meta, xs, w1, w3, w2)


P_G = 16


def _combine_body(ys_hbm, w0_hbm, pos_hbm, out_hbm,
                  i0_v, i1_v, g0_v, g1_v, out_v, w_v,
                  sg0a, sg0b, sg1a, sg1b, swa, swb):
    wid = lax.axis_index("s") * 2 + lax.axis_index("c")
    np_ = TPW // P_G
    sg0 = (sg0a, sg0b)
    sg1 = (sg1a, sg1b)
    sw = (swa, swb)
    wrcp = [None, None]

    def issue(p):
        b = p % 2
        base = wid * TPW + p * P_G
        pltpu.sync_copy(pos_hbm.at[wid, 0, pl.ds(p * P_G, P_G)], i0_v.at[b])
        pltpu.sync_copy(pos_hbm.at[wid, 1, pl.ds(p * P_G, P_G)], i1_v.at[b])
        c0 = pltpu.async_copy(ys_hbm.at[i0_v.at[b]], g0_v.at[b], sg0[b])
        c1 = pltpu.async_copy(ys_hbm.at[i1_v.at[b]], g1_v.at[b], sg1[b])
        pltpu.sync_copy(w0_hbm.at[pl.ds(base, P_G)], w_v.at[b])
        return (c0, c1)

    pend = issue(0)
    for p in range(np_):
        b = p % 2
        nxt = None
        if p + 1 < np_:
            nxt = issue(p + 1)
        c0, c1 = pend
        c0.wait()
        c1.wait()
        if wrcp[b] is not None:
            wrcp[b].wait()

        def row_body(r, carry):
            s0 = w_v[b, r]
            s1 = 1.0 - s0
            for c in range(D // 16):
                sl = pl.ds(c * 16, 16)
                out_v[b, r, sl] = (g0_v[b, r, sl] * s0
                                   + g1_v[b, r, sl] * s1)
            return carry

        lax.fori_loop(0, P_G, row_body, 0)
        wrcp[b] = pltpu.async_copy(
            out_v.at[b], out_hbm.at[pl.ds(wid * TPW + p * P_G, P_G)], sw[b])
        pend = nxt
    wrcp[0].wait()
    wrcp[1].wait()


@functools.lru_cache(maxsize=None)
def _make_combine_sc():
    return pl.kernel(
        _combine_body,
        out_type=jax.ShapeDtypeStruct((T, D), jnp.float32),
        mesh=_sc_mesh(),
        scratch_types=[
            pltpu.VMEM((2, P_G), jnp.int32),
            pltpu.VMEM((2, P_G), jnp.int32),
            pltpu.VMEM((2, P_G, D), jnp.float32),
            pltpu.VMEM((2, P_G, D), jnp.float32),
            pltpu.VMEM((2, P_G, D), jnp.float32),
            pltpu.VMEM((2, P_G, 16), jnp.float32),
            pltpu.SemaphoreType.DMA,
            pltpu.SemaphoreType.DMA,
            pltpu.SemaphoreType.DMA,
            pltpu.SemaphoreType.DMA,
            pltpu.SemaphoreType.DMA,
            pltpu.SemaphoreType.DMA,
        ],
    )


def _combine_sc(ys, w0b, pos):
    return _make_combine_sc()(ys, w0b, pos)


def _build_meta(counts, offs):
    m_grid = jnp.arange(NT_C, dtype=jnp.int32)[:, None]
    e_grid = jnp.arange(E, dtype=jnp.int32)[None, :]
    start = offs[None, :]
    end = (offs + counts)[None, :]
    lo = jnp.maximum(start, m_grid * TM_C)
    hi = jnp.minimum(end, (m_grid + 1) * TM_C)
    hit = (lo < hi).ravel()
    ar = jnp.arange(NT_C * E, dtype=jnp.int32)
    order = jnp.argsort(jnp.where(hit, ar, NT_C * E + ar))
    nh = jnp.sum(hit.astype(jnp.int32))
    src = order[jnp.minimum(jnp.arange(W_MAX, dtype=jnp.int32), nh - 1)]
    valid = jnp.arange(W_MAX, dtype=jnp.int32) < nh
    m_of = jnp.broadcast_to(m_grid, (NT_C, E)).ravel()[src]
    e_of = jnp.broadcast_to(e_grid, (NT_C, E)).ravel()[src]
    lo_j = jnp.where(valid, lo.ravel()[src], 0)
    hi_j = jnp.where(valid, hi.ravel()[src], 0)
    first = jnp.concatenate(
        [jnp.ones((1,), jnp.int32),
         (m_of[1:] != m_of[:-1]).astype(jnp.int32)])
    return jnp.stack([m_of, e_of, lo_j, hi_j, first]).astype(jnp.int32)


def kernel(hidden_states, Wr, W1, W3, W2, Ws1, Ws3, Ws2):
    b, s, d = hidden_states.shape
    x = hidden_states.reshape(T, D)
    logits, idx0c, idx1c, w0b, sel = _router(x, Wr)
    cfull, cg0, cg1 = _ranks(sel, idx0c, idx1c)
    counts = (cfull[-1] + sel[-1]).astype(jnp.int32)
    offs = jnp.concatenate(
        [jnp.zeros((1,), jnp.int32), jnp.cumsum(counts)[:-1]])
    idx0 = idx0c[:, 0]
    idx1 = idx1c[:, 0]
    pos0 = cg0[:, 0] + jnp.take(offs, idx0)
    pos1 = cg1[:, 0] + jnp.take(offs, idx1)
    pos = jnp.stack(
        [pos0.reshape(NW, TPW), pos1.reshape(NW, TPW)], axis=1)
    xs = _dispatch_sc(x, pos)
    meta = _build_meta(counts, offs)
    ys = _gmm(meta, xs, W1, W3, W2)
    ysh = _shared(x, Ws1, Ws3, Ws2)
    part = _combine_sc(ys, w0b, pos)
    out = _mix(part, ysh)
    return out.reshape(b, s, d), logits
